# baseline (speedup 1.0000x reference)
import functools
import math

import jax
import jax.numpy as jnp
from jax.experimental import pallas as pl

B, S, D, E, K = 4, 8192, 1024, 8, 2
N = B * S
EPS = 1e-10
_C0 = math.sqrt(2.0 / math.pi)

TBLK = 2048
NBLK = N // TBLK


def _router_body(x_ref, w1t_ref, b1_ref, w2t_ref, b2_ref, u_ref,
                 rp_ref, ti_ref, cnt_ref, loss_ref):
    i = pl.program_id(0)

    xb = x_ref[...]
    pre = jnp.dot(xb, w1t_ref[...], preferred_element_type=jnp.float32) + b1_ref[...]
    h = 0.5 * pre * (1.0 + jnp.tanh(_C0 * (pre + 0.044715 * pre * pre * pre)))
    logits = jnp.dot(h, w2t_ref[...], preferred_element_type=jnp.float32) + b2_ref[...]

    g = logits - jnp.log(-jnp.log(u_ref[...]))

    m = jnp.max(g, axis=1, keepdims=True)
    eg = jnp.exp(g - m)
    p = eg / jnp.sum(eg, axis=1, keepdims=True)

    idx8 = jax.lax.broadcasted_iota(jnp.int32, p.shape, 1)
    v1 = jnp.max(p, axis=1, keepdims=True)
    i1 = jnp.min(jnp.where(p == v1, idx8, E), axis=1, keepdims=True)
    pm = jnp.where(idx8 == i1, -1.0, p)
    v2 = jnp.max(pm, axis=1, keepdims=True)
    i2 = jnp.min(jnp.where(pm == v2, idx8, E), axis=1, keepdims=True)

    v1e = v1 + EPS
    v2e = v2 + EPS
    denom = jnp.maximum(v1e + v2e, 1e-12)
    rp_ref[...] = jnp.concatenate([v1e / denom, v2e / denom], axis=1)
    ti_ref[...] = jnp.concatenate([i1, i2], axis=1)

    onehot = (idx8 == i1).astype(jnp.float32) + (idx8 == i2).astype(jnp.float32)
    c = jnp.sum(onehot, axis=0, keepdims=True)

    @pl.when(i == 0)
    def _():
        cnt_ref[...] = c

    @pl.when(i != 0)
    def _():
        cnt_ref[...] += c

    @pl.when(i == NBLK - 1)
    def _():
        counts = cnt_ref[...]
        expert_probs = counts / jnp.sum(counts)
        t = jnp.float32(1.0 / E)
        kl = jnp.sum(t * (jnp.log(t) - jnp.log(expert_probs + EPS))) / E
        loss_ref[...] = jnp.full((1, 1), kl, dtype=jnp.float32)


@functools.partial(jax.jit, static_argnames=())
def kernel(x, W1, b1, W2, b2):
    x2 = x.reshape(N, D)
    gkey = jax.random.fold_in(jax.random.key(42), 7)
    u = jax.random.uniform(gkey, (B, S, E), minval=1e-20, maxval=1.0,
                           dtype=jnp.float32).reshape(N, E)

    grid = (NBLK,)
    rp, ti, cnt, loss = pl.pallas_call(
        _router_body,
        grid=grid,
        in_specs=[
            pl.BlockSpec((TBLK, D), lambda i: (i, 0)),
            pl.BlockSpec((D, E), lambda i: (0, 0)),
            pl.BlockSpec((1, E), lambda i: (0, 0)),
            pl.BlockSpec((E, E), lambda i: (0, 0)),
            pl.BlockSpec((1, E), lambda i: (0, 0)),
            pl.BlockSpec((TBLK, E), lambda i: (i, 0)),
        ],
        out_specs=[
            pl.BlockSpec((TBLK, K), lambda i: (i, 0)),
            pl.BlockSpec((TBLK, K), lambda i: (i, 0)),
            pl.BlockSpec((1, E), lambda i: (0, 0)),
            pl.BlockSpec((1, 1), lambda i: (0, 0)),
        ],
        out_shape=[
            jax.ShapeDtypeStruct((N, K), jnp.float32),
            jax.ShapeDtypeStruct((N, K), jnp.int32),
            jax.ShapeDtypeStruct((1, E), jnp.float32),
            jax.ShapeDtypeStruct((1, 1), jnp.float32),
        ],
    )(x2, W1.T, b1.reshape(1, E), W2.T, b2.reshape(1, E), u)

    router_probs = rp.reshape(B, S, K)
    top_k_indices = ti.reshape(B, S, K)
    expert_counts = cnt.reshape(E)
    load_balancing_loss = loss.reshape(())
    temperature = jnp.float32(1.0)
    return (router_probs, top_k_indices, load_balancing_loss, expert_counts,
            temperature)


# speedup vs baseline: 2.8984x; 2.8984x over previous
import functools
import math

import jax
import jax.numpy as jnp
from jax.experimental import pallas as pl

B, S, D, E, K = 4, 8192, 1024, 8, 2
N = B * S
EPS = 1e-10
_C0 = math.sqrt(2.0 / math.pi)

TBLK = 2048
NBLK = N // TBLK


def _router_body(x_ref, w1t_ref, b1_ref, w2t_ref, b2_ref, u_ref,
                 rp_ref, ti_ref, cnt_ref, loss_ref):
    i = pl.program_id(0)

    xb = x_ref[...]
    pre = jnp.dot(xb, w1t_ref[...], preferred_element_type=jnp.float32) + b1_ref[...]
    h = 0.5 * pre * (1.0 + jnp.tanh(_C0 * (pre + 0.044715 * pre * pre * pre)))
    logits = jnp.dot(h, w2t_ref[...], preferred_element_type=jnp.float32) + b2_ref[...]

    lt = logits.T
    g = lt - jnp.log(-jnp.log(u_ref[...]))

    m = jnp.max(g, axis=0, keepdims=True)
    eg = jnp.exp(g - m)
    p = eg / jnp.sum(eg, axis=0, keepdims=True)

    idx8 = jax.lax.broadcasted_iota(jnp.int32, p.shape, 0)
    v1 = jnp.max(p, axis=0, keepdims=True)
    i1 = jnp.min(jnp.where(p == v1, idx8, E), axis=0, keepdims=True)
    pm = jnp.where(idx8 == i1, -1.0, p)
    v2 = jnp.max(pm, axis=0, keepdims=True)
    i2 = jnp.min(jnp.where(pm == v2, idx8, E), axis=0, keepdims=True)

    v1e = v1 + EPS
    v2e = v2 + EPS
    denom = jnp.maximum(v1e + v2e, 1e-12)
    rp_ref[...] = jnp.concatenate([v1e / denom, v2e / denom], axis=0)
    ti_ref[...] = jnp.concatenate([i1, i2], axis=0)

    onehot = (idx8 == i1).astype(jnp.float32) + (idx8 == i2).astype(jnp.float32)
    c = jnp.sum(onehot, axis=1, keepdims=True)

    @pl.when(i == 0)
    def _():
        cnt_ref[...] = c

    @pl.when(i != 0)
    def _():
        cnt_ref[...] += c

    @pl.when(i == NBLK - 1)
    def _():
        counts = cnt_ref[...]
        expert_probs = counts / jnp.sum(counts)
        t = jnp.float32(1.0 / E)
        kl = jnp.sum(t * (jnp.log(t) - jnp.log(expert_probs + EPS))) / E
        loss_ref[...] = jnp.full((1, 1), kl, dtype=jnp.float32)


@functools.partial(jax.jit, static_argnames=())
def kernel(x, W1, b1, W2, b2):
    x2 = x.reshape(N, D)
    gkey = jax.random.fold_in(jax.random.key(42), 7)
    u = jax.random.uniform(gkey, (B, S, E), minval=1e-20, maxval=1.0,
                           dtype=jnp.float32).reshape(N, E).T

    grid = (NBLK,)
    rp, ti, cnt, loss = pl.pallas_call(
        _router_body,
        grid=grid,
        in_specs=[
            pl.BlockSpec((TBLK, D), lambda i: (i, 0)),
            pl.BlockSpec((D, E), lambda i: (0, 0)),
            pl.BlockSpec((1, E), lambda i: (0, 0)),
            pl.BlockSpec((E, E), lambda i: (0, 0)),
            pl.BlockSpec((1, E), lambda i: (0, 0)),
            pl.BlockSpec((E, TBLK), lambda i: (0, i)),
        ],
        out_specs=[
            pl.BlockSpec((K, TBLK), lambda i: (0, i)),
            pl.BlockSpec((K, TBLK), lambda i: (0, i)),
            pl.BlockSpec((E, 1), lambda i: (0, 0)),
            pl.BlockSpec((1, 1), lambda i: (0, 0)),
        ],
        out_shape=[
            jax.ShapeDtypeStruct((K, N), jnp.float32),
            jax.ShapeDtypeStruct((K, N), jnp.int32),
            jax.ShapeDtypeStruct((E, 1), jnp.float32),
            jax.ShapeDtypeStruct((1, 1), jnp.float32),
        ],
    )(x2, W1.T, b1.reshape(1, E), W2.T, b2.reshape(1, E), u)

    router_probs = rp.T.reshape(B, S, K)
    top_k_indices = ti.T.reshape(B, S, K)
    expert_counts = cnt.reshape(E)
    load_balancing_loss = loss.reshape(())
    temperature = jnp.float32(1.0)
    return (router_probs, top_k_indices, load_balancing_loss, expert_counts,
            temperature)


# speedup vs baseline: 2.9309x; 1.0112x over previous
import functools
import math

import jax
import jax.numpy as jnp
from jax.experimental import pallas as pl

B, S, D, E, K = 4, 8192, 1024, 8, 2
N = B * S
EPS = 1e-10
_C0 = math.sqrt(2.0 / math.pi)

TBLK = 4096
NBLK = N // TBLK


def _router_body(x_ref, w1t_ref, b1_ref, w2t_ref, b2_ref, u_ref,
                 rp_ref, ti_ref, cnt_ref, loss_ref):
    i = pl.program_id(0)

    xb = x_ref[...]
    pre = jnp.dot(xb, w1t_ref[...], preferred_element_type=jnp.float32) + b1_ref[...]
    h = 0.5 * pre * (1.0 + jnp.tanh(_C0 * (pre + 0.044715 * pre * pre * pre)))
    logits = jnp.dot(h, w2t_ref[...], preferred_element_type=jnp.float32) + b2_ref[...]

    lt = logits.T
    g = lt - jnp.log(-jnp.log(u_ref[...]))

    m = jnp.max(g, axis=0, keepdims=True)
    eg = jnp.exp(g - m)
    p = eg / jnp.sum(eg, axis=0, keepdims=True)

    idx8 = jax.lax.broadcasted_iota(jnp.int32, p.shape, 0)
    v1 = jnp.max(p, axis=0, keepdims=True)
    i1 = jnp.min(jnp.where(p == v1, idx8, E), axis=0, keepdims=True)
    pm = jnp.where(idx8 == i1, -1.0, p)
    v2 = jnp.max(pm, axis=0, keepdims=True)
    i2 = jnp.min(jnp.where(pm == v2, idx8, E), axis=0, keepdims=True)

    v1e = v1 + EPS
    v2e = v2 + EPS
    denom = jnp.maximum(v1e + v2e, 1e-12)
    rp_ref[...] = jnp.concatenate([v1e / denom, v2e / denom], axis=0)
    ti_ref[...] = jnp.concatenate([i1, i2], axis=0)

    onehot = (idx8 == i1).astype(jnp.float32) + (idx8 == i2).astype(jnp.float32)
    c = jnp.sum(onehot, axis=1, keepdims=True)

    @pl.when(i == 0)
    def _():
        cnt_ref[...] = c

    @pl.when(i != 0)
    def _():
        cnt_ref[...] += c

    @pl.when(i == NBLK - 1)
    def _():
        counts = cnt_ref[...]
        expert_probs = counts / jnp.sum(counts)
        t = jnp.float32(1.0 / E)
        kl = jnp.sum(t * (jnp.log(t) - jnp.log(expert_probs + EPS))) / E
        loss_ref[...] = jnp.full((1, 1), kl, dtype=jnp.float32)


@functools.partial(jax.jit, static_argnames=())
def kernel(x, W1, b1, W2, b2):
    x2 = x.reshape(N, D)
    gkey = jax.random.fold_in(jax.random.key(42), 7)
    u = jax.random.uniform(gkey, (B, S, E), minval=1e-20, maxval=1.0,
                           dtype=jnp.float32).reshape(N, E).T

    grid = (NBLK,)
    rp, ti, cnt, loss = pl.pallas_call(
        _router_body,
        grid=grid,
        in_specs=[
            pl.BlockSpec((TBLK, D), lambda i: (i, 0)),
            pl.BlockSpec((D, E), lambda i: (0, 0)),
            pl.BlockSpec((1, E), lambda i: (0, 0)),
            pl.BlockSpec((E, E), lambda i: (0, 0)),
            pl.BlockSpec((1, E), lambda i: (0, 0)),
            pl.BlockSpec((E, TBLK), lambda i: (0, i)),
        ],
        out_specs=[
            pl.BlockSpec((K, TBLK), lambda i: (0, i)),
            pl.BlockSpec((K, TBLK), lambda i: (0, i)),
            pl.BlockSpec((E, 1), lambda i: (0, 0)),
            pl.BlockSpec((1, 1), lambda i: (0, 0)),
        ],
        out_shape=[
            jax.ShapeDtypeStruct((K, N), jnp.float32),
            jax.ShapeDtypeStruct((K, N), jnp.int32),
            jax.ShapeDtypeStruct((E, 1), jnp.float32),
            jax.ShapeDtypeStruct((1, 1), jnp.float32),
        ],
    )(x2, W1.T, b1.reshape(1, E), W2.T, b2.reshape(1, E), u)

    router_probs = rp.T.reshape(B, S, K)
    top_k_indices = ti.T.reshape(B, S, K)
    expert_counts = cnt.reshape(E)
    load_balancing_loss = loss.reshape(())
    temperature = jnp.float32(1.0)
    return (router_probs, top_k_indices, load_balancing_loss, expert_counts,
            temperature)
